# baseline (device time: 201131 ns/iter reference)
import jax
import jax.numpy as jnp
from jax import lax
from jax.experimental import pallas as pl
from jax.experimental.pallas import tpu as pltpu

N_DEV = 4
SQ = 2048
SKV = 2048
D_MODEL = 1024
H_PER = 8
DH = 128
BQ = 512
SCALE = 0.08838834764831843
BLK = 64


def _qproj_body(x_ref, wq_ref, q_ref):
    x = x_ref[...].astype(jnp.bfloat16)
    wq = wq_ref[...].astype(jnp.bfloat16)
    q_ref[...] = (
        jnp.dot(x, wq, preferred_element_type=jnp.float32) * SCALE
    ).astype(jnp.bfloat16)


BKV = 512


def _attn_body(idx_ref, q_ref, k_ref, v_ref, ctx_ref):
    qi = pl.program_id(1)
    q = q_ref[...]

    def step(j, carry):
        acc, l = carry
        k = k_ref[pl.ds(j * BKV, BKV), :].astype(jnp.bfloat16)
        s = lax.dot_general(
            q, k, (((1,), (1,)), ((), ())),
            preferred_element_type=jnp.float32,
        )
        p = jnp.exp(s)
        l = l + jnp.sum(p, axis=1, keepdims=True)
        v = v_ref[pl.ds(j * BKV, BKV), :].astype(jnp.bfloat16)
        acc = acc + jnp.dot(
            p.astype(jnp.bfloat16), v, preferred_element_type=jnp.float32
        )
        return acc, l

    init = (
        jnp.zeros((BQ, DH), jnp.float32),
        jnp.zeros((BQ, 1), jnp.float32),
    )
    acc, l = lax.fori_loop(0, qi, init_val=init, body_fun=step)

    k = k_ref[pl.ds(qi * BKV, BKV), :].astype(jnp.bfloat16)
    s = lax.dot_general(
        q, k, (((1,), (1,)), ((), ())), preferred_element_type=jnp.float32
    )
    rowb = lax.broadcasted_iota(jnp.int32, (BQ, BKV), 0) // BLK
    colb = lax.broadcasted_iota(jnp.int32, (BQ, BKV), 1) // BLK
    p = jnp.where(colb <= rowb, jnp.exp(s), 0.0)
    l = l + jnp.sum(p, axis=1, keepdims=True)
    v = v_ref[pl.ds(qi * BKV, BKV), :].astype(jnp.bfloat16)
    acc = acc + jnp.dot(
        p.astype(jnp.bfloat16), v, preferred_element_type=jnp.float32
    )
    ctx_ref[...] = (acc / l).astype(jnp.bfloat16)


def _oproj_body(ctx_ref, wo_ref, p_ref):
    ctx = ctx_ref[...]
    wo = wo_ref[...].astype(jnp.bfloat16)
    p_ref[...] = jnp.dot(
        ctx, wo, preferred_element_type=jnp.float32
    ).astype(jnp.bfloat16)


HALF = SQ // 2


def _allreduce_body(p_ref, out_ref, sbuf_ref, rbuf1_ref, rbuf2_ref,
                    send_sems, recv_sems):
    my = lax.axis_index("i")
    partner_a = my ^ 1
    partner_b = 3 - my

    barrier = pltpu.get_barrier_semaphore()
    for nbr in (partner_a, partner_b):
        pl.semaphore_signal(
            barrier, inc=1, device_id=(nbr,),
            device_id_type=pl.DeviceIdType.MESH,
        )
    pl.semaphore_wait(barrier, 2)

    r1a = pltpu.make_async_remote_copy(
        src_ref=p_ref.at[pl.ds(0, HALF)],
        dst_ref=rbuf1_ref.at[0],
        send_sem=send_sems.at[0, 0],
        recv_sem=recv_sems.at[0, 0],
        device_id=(partner_a,),
        device_id_type=pl.DeviceIdType.MESH,
    )
    r1b = pltpu.make_async_remote_copy(
        src_ref=p_ref.at[pl.ds(HALF, HALF)],
        dst_ref=rbuf1_ref.at[1],
        send_sem=send_sems.at[0, 1],
        recv_sem=recv_sems.at[0, 1],
        device_id=(partner_b,),
        device_id_type=pl.DeviceIdType.MESH,
    )
    r1a.start()
    r1b.start()
    r1a.wait()
    r1b.wait()

    sbuf_ref[0] = p_ref[pl.ds(0, HALF)] + rbuf1_ref[0]
    sbuf_ref[1] = p_ref[pl.ds(HALF, HALF)] + rbuf1_ref[1]

    r2a = pltpu.make_async_remote_copy(
        src_ref=sbuf_ref.at[0],
        dst_ref=rbuf2_ref.at[0],
        send_sem=send_sems.at[1, 0],
        recv_sem=recv_sems.at[1, 0],
        device_id=(partner_b,),
        device_id_type=pl.DeviceIdType.MESH,
    )
    r2b = pltpu.make_async_remote_copy(
        src_ref=sbuf_ref.at[1],
        dst_ref=rbuf2_ref.at[1],
        send_sem=send_sems.at[1, 1],
        recv_sem=recv_sems.at[1, 1],
        device_id=(partner_a,),
        device_id_type=pl.DeviceIdType.MESH,
    )
    r2a.start()
    r2b.start()
    r2a.wait()
    r2b.wait()

    out_ref[pl.ds(0, HALF)] = (
        sbuf_ref[0].astype(jnp.float32) + rbuf2_ref[0].astype(jnp.float32)
    )
    out_ref[pl.ds(HALF, HALF)] = (
        sbuf_ref[1].astype(jnp.float32) + rbuf2_ref[1].astype(jnp.float32)
    )


def kernel(x, Wq, K_ext, V_ext, Wo):
    my = lax.axis_index("i")
    x2 = x.reshape(SQ, D_MODEL)
    K = K_ext.reshape(SKV, 32 * DH)
    V = V_ext.reshape(SKV, 32 * DH)
    my_idx = my.astype(jnp.int32).reshape((1,))

    Q = pl.pallas_call(
        _qproj_body,
        out_shape=jax.ShapeDtypeStruct((SQ, D_MODEL), jnp.bfloat16),
        in_specs=[
            pl.BlockSpec(memory_space=pltpu.VMEM),
            pl.BlockSpec(memory_space=pltpu.VMEM),
        ],
        out_specs=pl.BlockSpec(memory_space=pltpu.VMEM),
    )(x2, Wq)

    ctx = pl.pallas_call(
        _attn_body,
        grid_spec=pltpu.PrefetchScalarGridSpec(
            num_scalar_prefetch=1,
            grid=(H_PER, SQ // BQ),
            in_specs=[
                pl.BlockSpec((BQ, DH), lambda h, qi, s: (qi, h)),
                pl.BlockSpec((SKV, DH), lambda h, qi, s: (0, s[0] * H_PER + h)),
                pl.BlockSpec((SKV, DH), lambda h, qi, s: (0, s[0] * H_PER + h)),
            ],
            out_specs=pl.BlockSpec((BQ, DH), lambda h, qi, s: (qi, h)),
        ),
        out_shape=jax.ShapeDtypeStruct((SQ, H_PER * DH), jnp.bfloat16),
    )(my_idx, Q, K, V)

    partial = pl.pallas_call(
        _oproj_body,
        out_shape=jax.ShapeDtypeStruct((SQ, D_MODEL), jnp.bfloat16),
        in_specs=[
            pl.BlockSpec(memory_space=pltpu.VMEM),
            pl.BlockSpec(memory_space=pltpu.VMEM),
        ],
        out_specs=pl.BlockSpec(memory_space=pltpu.VMEM),
    )(ctx, Wo)

    out = pl.pallas_call(
        _allreduce_body,
        out_shape=jax.ShapeDtypeStruct((SQ, D_MODEL), jnp.float32),
        in_specs=[pl.BlockSpec(memory_space=pltpu.VMEM)],
        out_specs=pl.BlockSpec(memory_space=pltpu.VMEM),
        scratch_shapes=[
            pltpu.VMEM((2, HALF, D_MODEL), jnp.bfloat16),
            pltpu.VMEM((2, HALF, D_MODEL), jnp.bfloat16),
            pltpu.VMEM((2, HALF, D_MODEL), jnp.bfloat16),
            pltpu.SemaphoreType.DMA((2, 2)),
            pltpu.SemaphoreType.DMA((2, 2)),
        ],
        compiler_params=pltpu.CompilerParams(collective_id=0),
    )(partial)

    return out.reshape(1, SQ, D_MODEL)


# device time: 132826 ns/iter; 1.5142x vs baseline; 1.5142x over previous
import jax
import jax.numpy as jnp
from jax import lax
from jax.experimental import pallas as pl
from jax.experimental.pallas import tpu as pltpu

N_DEV = 4
SQ = 2048
SKV = 2048
D_MODEL = 1024
H_PER = 8
DH = 128
BQ = 512
SCALE = 0.08838834764831843
BLK = 64


def _qproj_body(x_ref, wq_ref, q_ref):
    x = x_ref[...].astype(jnp.bfloat16)
    wq = wq_ref[...].astype(jnp.bfloat16)
    q_ref[...] = (
        jnp.dot(x, wq, preferred_element_type=jnp.float32) * SCALE
    ).astype(jnp.bfloat16)


BKV = 512


def _attn_body(q_ref, k_ref, v_ref, ctx_ref):
    qi = pl.program_id(1)
    q = q_ref[...]

    def step(j, carry):
        acc, l = carry
        k = k_ref[0, pl.ds(j * BKV, BKV), :]
        s = lax.dot_general(
            q, k, (((1,), (1,)), ((), ())),
            preferred_element_type=jnp.float32,
        )
        p = jnp.exp(s)
        l = l + jnp.sum(p, axis=1, keepdims=True)
        v = v_ref[0, pl.ds(j * BKV, BKV), :]
        acc = acc + jnp.dot(
            p.astype(jnp.bfloat16), v, preferred_element_type=jnp.float32
        )
        return acc, l

    init = (
        jnp.zeros((BQ, DH), jnp.float32),
        jnp.zeros((BQ, 1), jnp.float32),
    )
    acc, l = lax.fori_loop(0, qi, init_val=init, body_fun=step)

    k = k_ref[0, pl.ds(qi * BKV, BKV), :]
    s = lax.dot_general(
        q, k, (((1,), (1,)), ((), ())), preferred_element_type=jnp.float32
    )
    rowb = lax.broadcasted_iota(jnp.int32, (BQ, BKV), 0) // BLK
    colb = lax.broadcasted_iota(jnp.int32, (BQ, BKV), 1) // BLK
    p = jnp.where(colb <= rowb, jnp.exp(s), 0.0)
    l = l + jnp.sum(p, axis=1, keepdims=True)
    v = v_ref[0, pl.ds(qi * BKV, BKV), :]
    acc = acc + jnp.dot(
        p.astype(jnp.bfloat16), v, preferred_element_type=jnp.float32
    )
    ctx_ref[...] = (acc / l).astype(jnp.bfloat16)


def _oproj_body(ctx_ref, wo_ref, p_ref):
    ctx = ctx_ref[...]
    wo = wo_ref[...].astype(jnp.bfloat16)
    p_ref[...] = jnp.dot(
        ctx, wo, preferred_element_type=jnp.float32
    ).astype(jnp.bfloat16)


HALF = SQ // 2


def _allreduce_body(p_ref, out_ref, sbuf_ref, rbuf1_ref, rbuf2_ref,
                    send_sems, recv_sems):
    my = lax.axis_index("i")
    partner_a = my ^ 1
    partner_b = 3 - my

    barrier = pltpu.get_barrier_semaphore()
    for nbr in (partner_a, partner_b):
        pl.semaphore_signal(
            barrier, inc=1, device_id=(nbr,),
            device_id_type=pl.DeviceIdType.MESH,
        )
    pl.semaphore_wait(barrier, 2)

    r1a = pltpu.make_async_remote_copy(
        src_ref=p_ref.at[pl.ds(0, HALF)],
        dst_ref=rbuf1_ref.at[0],
        send_sem=send_sems.at[0, 0],
        recv_sem=recv_sems.at[0, 0],
        device_id=(partner_a,),
        device_id_type=pl.DeviceIdType.MESH,
    )
    r1b = pltpu.make_async_remote_copy(
        src_ref=p_ref.at[pl.ds(HALF, HALF)],
        dst_ref=rbuf1_ref.at[1],
        send_sem=send_sems.at[0, 1],
        recv_sem=recv_sems.at[0, 1],
        device_id=(partner_b,),
        device_id_type=pl.DeviceIdType.MESH,
    )
    r1a.start()
    r1b.start()
    r1a.wait()
    r1b.wait()

    sbuf_ref[0] = p_ref[pl.ds(0, HALF)] + rbuf1_ref[0]
    sbuf_ref[1] = p_ref[pl.ds(HALF, HALF)] + rbuf1_ref[1]

    r2a = pltpu.make_async_remote_copy(
        src_ref=sbuf_ref.at[0],
        dst_ref=rbuf2_ref.at[0],
        send_sem=send_sems.at[1, 0],
        recv_sem=recv_sems.at[1, 0],
        device_id=(partner_b,),
        device_id_type=pl.DeviceIdType.MESH,
    )
    r2b = pltpu.make_async_remote_copy(
        src_ref=sbuf_ref.at[1],
        dst_ref=rbuf2_ref.at[1],
        send_sem=send_sems.at[1, 1],
        recv_sem=recv_sems.at[1, 1],
        device_id=(partner_a,),
        device_id_type=pl.DeviceIdType.MESH,
    )
    r2a.start()
    r2b.start()
    r2a.wait()
    r2b.wait()

    out_ref[pl.ds(0, HALF)] = (
        sbuf_ref[0].astype(jnp.float32) + rbuf2_ref[0].astype(jnp.float32)
    )
    out_ref[pl.ds(HALF, HALF)] = (
        sbuf_ref[1].astype(jnp.float32) + rbuf2_ref[1].astype(jnp.float32)
    )


def kernel(x, Wq, K_ext, V_ext, Wo):
    my = lax.axis_index("i")
    x2 = x.reshape(SQ, D_MODEL)
    K = lax.dynamic_slice_in_dim(
        K_ext.reshape(SKV, 32, DH), my * H_PER, H_PER, axis=1
    ).astype(jnp.bfloat16).transpose(1, 0, 2)
    V = lax.dynamic_slice_in_dim(
        V_ext.reshape(SKV, 32, DH), my * H_PER, H_PER, axis=1
    ).astype(jnp.bfloat16).transpose(1, 0, 2)

    Q = pl.pallas_call(
        _qproj_body,
        out_shape=jax.ShapeDtypeStruct((SQ, D_MODEL), jnp.bfloat16),
        in_specs=[
            pl.BlockSpec(memory_space=pltpu.VMEM),
            pl.BlockSpec(memory_space=pltpu.VMEM),
        ],
        out_specs=pl.BlockSpec(memory_space=pltpu.VMEM),
    )(x2, Wq)

    ctx = pl.pallas_call(
        _attn_body,
        grid=(H_PER, SQ // BQ),
        out_shape=jax.ShapeDtypeStruct((SQ, H_PER * DH), jnp.bfloat16),
        in_specs=[
            pl.BlockSpec((BQ, DH), lambda h, qi: (qi, h)),
            pl.BlockSpec((1, SKV, DH), lambda h, qi: (h, 0, 0)),
            pl.BlockSpec((1, SKV, DH), lambda h, qi: (h, 0, 0)),
        ],
        out_specs=pl.BlockSpec((BQ, DH), lambda h, qi: (qi, h)),
    )(Q, K, V)

    partial = pl.pallas_call(
        _oproj_body,
        out_shape=jax.ShapeDtypeStruct((SQ, D_MODEL), jnp.bfloat16),
        in_specs=[
            pl.BlockSpec(memory_space=pltpu.VMEM),
            pl.BlockSpec(memory_space=pltpu.VMEM),
        ],
        out_specs=pl.BlockSpec(memory_space=pltpu.VMEM),
    )(ctx, Wo)

    out = pl.pallas_call(
        _allreduce_body,
        out_shape=jax.ShapeDtypeStruct((SQ, D_MODEL), jnp.float32),
        in_specs=[pl.BlockSpec(memory_space=pltpu.VMEM)],
        out_specs=pl.BlockSpec(memory_space=pltpu.VMEM),
        scratch_shapes=[
            pltpu.VMEM((2, HALF, D_MODEL), jnp.bfloat16),
            pltpu.VMEM((2, HALF, D_MODEL), jnp.bfloat16),
            pltpu.VMEM((2, HALF, D_MODEL), jnp.bfloat16),
            pltpu.SemaphoreType.DMA((2, 2)),
            pltpu.SemaphoreType.DMA((2, 2)),
        ],
        compiler_params=pltpu.CompilerParams(collective_id=0),
    )(partial)

    return out.reshape(1, SQ, D_MODEL)


# device time: 89825 ns/iter; 2.2391x vs baseline; 1.4787x over previous
import jax
import jax.numpy as jnp
from jax import lax
from jax.experimental import pallas as pl
from jax.experimental.pallas import tpu as pltpu

N_DEV = 4
SQ = 2048
SKV = 2048
D_MODEL = 1024
H_PER = 8
DH = 128
BQ = 512
NB = SQ // BQ
QH = BQ // 2
SCALE = 0.08838834764831843
BLK = 64


def _qproj_body(x_ref, wq_ref, q_ref):
    x = x_ref[...].astype(jnp.bfloat16)
    wq = wq_ref[...].astype(jnp.bfloat16)
    q_ref[...] = (
        jnp.dot(x, wq, preferred_element_type=jnp.float32) * SCALE
    ).astype(jnp.bfloat16)


def _mega_body(q_ref, k_ref, v_ref, wo_ref, out_ref,
               ctx_ref, p_ref, sbuf_ref, rbuf1_ref, rbuf2_ref,
               s1_send, s1_recv, s2_send, s2_recv):
    my = lax.axis_index("i")
    partner_a = my ^ 1
    partner_b = 3 - my

    barrier = pltpu.get_barrier_semaphore()
    for nbr in (partner_a, partner_b):
        pl.semaphore_signal(
            barrier, inc=1, device_id=(nbr,),
            device_id_type=pl.DeviceIdType.MESH,
        )
    pl.semaphore_wait(barrier, 2)

    wo = wo_ref[...].astype(jnp.bfloat16)

    stage1 = {}
    stage2 = {}

    def start_stage1(qi):
        r0 = qi * BQ
        ra = pltpu.make_async_remote_copy(
            src_ref=p_ref.at[pl.ds(r0, QH), :],
            dst_ref=rbuf1_ref.at[qi, 0],
            send_sem=s1_send.at[qi, 0],
            recv_sem=s1_recv.at[qi, 0],
            device_id=(partner_a,),
            device_id_type=pl.DeviceIdType.MESH,
        )
        rb = pltpu.make_async_remote_copy(
            src_ref=p_ref.at[pl.ds(r0 + QH, QH), :],
            dst_ref=rbuf1_ref.at[qi, 1],
            send_sem=s1_send.at[qi, 1],
            recv_sem=s1_recv.at[qi, 1],
            device_id=(partner_b,),
            device_id_type=pl.DeviceIdType.MESH,
        )
        ra.start()
        rb.start()
        stage1[qi] = (ra, rb)

    def finish1_start2(qi):
        ra, rb = stage1[qi]
        ra.wait()
        rb.wait()
        r0 = qi * BQ
        sbuf_ref[qi, 0] = p_ref[pl.ds(r0, QH), :] + rbuf1_ref[qi, 0]
        sbuf_ref[qi, 1] = p_ref[pl.ds(r0 + QH, QH), :] + rbuf1_ref[qi, 1]
        ra2 = pltpu.make_async_remote_copy(
            src_ref=sbuf_ref.at[qi, 0],
            dst_ref=rbuf2_ref.at[qi, 0],
            send_sem=s2_send.at[qi, 0],
            recv_sem=s2_recv.at[qi, 0],
            device_id=(partner_b,),
            device_id_type=pl.DeviceIdType.MESH,
        )
        rb2 = pltpu.make_async_remote_copy(
            src_ref=sbuf_ref.at[qi, 1],
            dst_ref=rbuf2_ref.at[qi, 1],
            send_sem=s2_send.at[qi, 1],
            recv_sem=s2_recv.at[qi, 1],
            device_id=(partner_a,),
            device_id_type=pl.DeviceIdType.MESH,
        )
        ra2.start()
        rb2.start()
        stage2[qi] = (ra2, rb2)

    for qi in range(NB):
        r0 = qi * BQ
        for h in range(H_PER):
            q = q_ref[pl.ds(r0, BQ), pl.ds(h * DH, DH)]
            l = jnp.zeros((BQ, 1), jnp.float32)
            acc = jnp.zeros((BQ, DH), jnp.float32)
            for j in range(qi):
                kc = k_ref[h, pl.ds(j * BQ, BQ), :]
                sc = lax.dot_general(
                    q, kc, (((1,), (1,)), ((), ())),
                    preferred_element_type=jnp.float32,
                )
                pc = jnp.exp(sc)
                l = l + jnp.sum(pc, axis=1, keepdims=True)
                acc = acc + jnp.dot(
                    pc.astype(jnp.bfloat16),
                    v_ref[h, pl.ds(j * BQ, BQ), :],
                    preferred_element_type=jnp.float32,
                )
            kd = k_ref[h, pl.ds(r0, BQ), :]
            s_d = lax.dot_general(
                q, kd, (((1,), (1,)), ((), ())),
                preferred_element_type=jnp.float32,
            )
            rowb = lax.broadcasted_iota(jnp.int32, (BQ, BQ), 0) // BLK
            colb = lax.broadcasted_iota(jnp.int32, (BQ, BQ), 1) // BLK
            p_d = jnp.where(colb <= rowb, jnp.exp(s_d), 0.0)
            l = l + jnp.sum(p_d, axis=1, keepdims=True)
            acc = acc + jnp.dot(
                p_d.astype(jnp.bfloat16),
                v_ref[h, pl.ds(r0, BQ), :],
                preferred_element_type=jnp.float32,
            )
            ctx_ref[:, pl.ds(h * DH, DH)] = (acc / l).astype(jnp.bfloat16)

        p_ref[pl.ds(r0, BQ), :] = jnp.dot(
            ctx_ref[...], wo, preferred_element_type=jnp.float32
        ).astype(jnp.bfloat16)
        start_stage1(qi)
        if qi > 0:
            finish1_start2(qi - 1)

    finish1_start2(NB - 1)

    for qi in range(NB):
        ra2, rb2 = stage2[qi]
        ra2.wait()
        rb2.wait()
        r0 = qi * BQ
        out_ref[pl.ds(r0, QH), :] = (
            sbuf_ref[qi, 0].astype(jnp.float32)
            + rbuf2_ref[qi, 0].astype(jnp.float32)
        )
        out_ref[pl.ds(r0 + QH, QH), :] = (
            sbuf_ref[qi, 1].astype(jnp.float32)
            + rbuf2_ref[qi, 1].astype(jnp.float32)
        )


def kernel(x, Wq, K_ext, V_ext, Wo):
    my = lax.axis_index("i")
    x2 = x.reshape(SQ, D_MODEL)
    K = lax.dynamic_slice_in_dim(
        K_ext.reshape(SKV, 32, DH), my * H_PER, H_PER, axis=1
    ).astype(jnp.bfloat16).transpose(1, 0, 2)
    V = lax.dynamic_slice_in_dim(
        V_ext.reshape(SKV, 32, DH), my * H_PER, H_PER, axis=1
    ).astype(jnp.bfloat16).transpose(1, 0, 2)

    Q = pl.pallas_call(
        _qproj_body,
        out_shape=jax.ShapeDtypeStruct((SQ, D_MODEL), jnp.bfloat16),
        in_specs=[
            pl.BlockSpec(memory_space=pltpu.VMEM),
            pl.BlockSpec(memory_space=pltpu.VMEM),
        ],
        out_specs=pl.BlockSpec(memory_space=pltpu.VMEM),
    )(x2, Wq)

    out = pl.pallas_call(
        _mega_body,
        out_shape=jax.ShapeDtypeStruct((SQ, D_MODEL), jnp.float32),
        in_specs=[
            pl.BlockSpec(memory_space=pltpu.VMEM),
            pl.BlockSpec(memory_space=pltpu.VMEM),
            pl.BlockSpec(memory_space=pltpu.VMEM),
            pl.BlockSpec(memory_space=pltpu.VMEM),
        ],
        out_specs=pl.BlockSpec(memory_space=pltpu.VMEM),
        scratch_shapes=[
            pltpu.VMEM((BQ, H_PER * DH), jnp.bfloat16),
            pltpu.VMEM((SQ, D_MODEL), jnp.bfloat16),
            pltpu.VMEM((NB, 2, QH, D_MODEL), jnp.bfloat16),
            pltpu.VMEM((NB, 2, QH, D_MODEL), jnp.bfloat16),
            pltpu.VMEM((NB, 2, QH, D_MODEL), jnp.bfloat16),
            pltpu.SemaphoreType.DMA((NB, 2)),
            pltpu.SemaphoreType.DMA((NB, 2)),
            pltpu.SemaphoreType.DMA((NB, 2)),
            pltpu.SemaphoreType.DMA((NB, 2)),
        ],
        compiler_params=pltpu.CompilerParams(collective_id=0),
    )(Q, K, V, Wo)

    return out.reshape(1, SQ, D_MODEL)
